# baseline (speedup 1.0000x reference)
import functools

import jax
import jax.numpy as jnp
from jax import lax
from jax.experimental import pallas as pl
from jax.experimental.pallas import tpu as pltpu
from jax.experimental.pallas import tpu_sc as plsc

N = 50000
D = 128
K = 1000
NW = 32
CH = 112
NCH = 14
PERW = CH * NCH
NPAD = NW * PERW
B = 1000
GRID = N // B
B2 = 1024
GRID2 = 49
NEG = -1e30


def _dot(a, b):
    return lax.dot_general(a, b, (((1,), (0,)), ((), ())),
                           preferred_element_type=jnp.float32)


def _dott(a, b):
    return lax.dot_general(a, b, (((1,), (1,)), ((), ())),
                           preferred_element_type=jnp.float32)


KP = 1024


def _prep_body(ct_ref, watt_ref, wpdt_ref, bpdt_ref, wcomb_ref, bcomb_ref,
               s2_ref, g_ref, m_ref, c0_ref):
    ctp = ct_ref[...]
    ct = ctp[:K, :]
    wcomb = wcomb_ref[...]
    wc_p = wcomb[:, :D]
    wc_c = wcomb[:, D:]
    wa_c = watt_ref[:, D:]
    s_row = _dott(wa_c, ctp)
    for h in range(8):
        s2_ref[h:h + 1, :] = s_row[:, h * 128:(h + 1) * 128]
    g_ref[...] = _dott(ct, wc_c)
    m_ref[...] = lax.dot_general(wpdt_ref[...], wc_p, (((0,), (1,)), ((), ())),
                                 preferred_element_type=jnp.float32)
    c0_ref[...] = _dott(bpdt_ref[...], wc_p) + bcomb_ref[...]


_prep = pl.pallas_call(
    _prep_body,
    in_specs=[
        pl.BlockSpec((KP, D), lambda: (0, 0)),
        pl.BlockSpec((1, 2 * D), lambda: (0, 0)),
        pl.BlockSpec((D, D), lambda: (0, 0)),
        pl.BlockSpec((1, D), lambda: (0, 0)),
        pl.BlockSpec((D, 2 * D), lambda: (0, 0)),
        pl.BlockSpec((1, D), lambda: (0, 0)),
    ],
    out_specs=[
        pl.BlockSpec((8, 128), lambda: (0, 0)),
        pl.BlockSpec((K, D), lambda: (0, 0)),
        pl.BlockSpec((D, D), lambda: (0, 0)),
        pl.BlockSpec((1, D), lambda: (0, 0)),
    ],
    out_shape=[
        jax.ShapeDtypeStruct((8, 128), jnp.float32),
        jax.ShapeDtypeStruct((K, D), jnp.float32),
        jax.ShapeDtypeStruct((D, D), jnp.float32),
        jax.ShapeDtypeStruct((1, D), jnp.float32),
    ],
)


def _sc_gather_body(idx_hbm, tab_hbm, out_hbm, idx_v, rows0, rows1, sem0, sem1):
    wid = lax.axis_index("s") * 2 + lax.axis_index("c")
    base = wid * PERW
    pltpu.sync_copy(idx_hbm.at[wid], idx_v)
    bufs = (rows0, rows1)
    sems = (sem0, sem1)
    copies = [
        pltpu.make_async_copy(tab_hbm.at[idx_v.at[j]], bufs[j % 2], sems[j % 2])
        for j in range(NCH)
    ]
    copies[0].start()
    for j in range(NCH):
        if j + 1 < NCH:
            copies[j + 1].start()
        copies[j].wait()
        pltpu.sync_copy(bufs[j % 2], out_hbm.at[pl.ds(base + j * CH, CH)])


@functools.lru_cache(maxsize=None)
def _make_sc_gather():
    mesh = plsc.VectorSubcoreMesh(core_axis_name="c", subcore_axis_name="s")
    return pl.kernel(
        _sc_gather_body,
        out_type=jax.ShapeDtypeStruct((NPAD, D), jnp.float32),
        mesh=mesh,
        scratch_types=[
            pltpu.VMEM((NCH, CH), jnp.int32),
            pltpu.VMEM((CH, D), jnp.float32),
            pltpu.VMEM((CH, D), jnp.float32),
            pltpu.SemaphoreType.DMA,
            pltpu.SemaphoreType.DMA,
        ],
    )


def _pass1_body(pf_ref, e_ref, s2_ref, watt_ref, scores_ref, stats_ref,
                acc_ref):
    pid = pl.program_id(0)

    @pl.when(pid == 0)
    def _():
        acc_ref[0] = NEG
        acc_ref[1] = 0.0

    pf = pf_ref[...]
    wa_p = watt_ref[:, :D]
    u_col = jnp.sum(pf * wa_p, axis=1, keepdims=True)
    e_row = e_ref[...].reshape(1, B)
    hi = e_row >> 7
    lo = e_row & 127
    oh_hi = (lax.broadcasted_iota(jnp.int32, (8, B), 0) == hi
             ).astype(jnp.float32)
    s_sel = lax.dot_general(s2_ref[...], oh_hi, (((0,), (0,)), ((), ())),
                            preferred_element_type=jnp.float32)
    oh_lo = (lax.broadcasted_iota(jnp.int32, (128, B), 0) == lo
             ).astype(jnp.float32)
    x = oh_lo * s_sel
    ones = jnp.ones((128, 128), jnp.float32)
    s_e = lax.dot_general(x, ones, (((0,), (0,)), ((), ())),
                          preferred_element_type=jnp.float32)
    sc = u_col + s_e[:, :1]
    scores_ref[...] = sc

    m_old = acc_ref[0]
    z_old = acc_ref[1]
    m_new = jnp.maximum(m_old, jnp.max(sc))
    z_new = z_old * jnp.exp(m_old - m_new) + jnp.sum(jnp.exp(sc - m_new))
    acc_ref[0] = m_new
    acc_ref[1] = z_new

    @pl.when(pid == GRID - 1)
    def _():
        stats_ref[0] = m_new
        stats_ref[1] = z_new


_pass1 = pl.pallas_call(
    _pass1_body,
    grid=(GRID,),
    in_specs=[
        pl.BlockSpec((B, D), lambda i: (i, 0)),
        pl.BlockSpec((1, 1, B), lambda i: (i, 0, 0)),
        pl.BlockSpec((8, 128), lambda i: (0, 0)),
        pl.BlockSpec((1, 2 * D), lambda i: (0, 0)),
    ],
    out_specs=[
        pl.BlockSpec((B, 1), lambda i: (i, 0)),
        pl.BlockSpec(memory_space=pltpu.SMEM),
    ],
    out_shape=[
        jax.ShapeDtypeStruct((N, 1), jnp.float32),
        jax.ShapeDtypeStruct((2,), jnp.float32),
    ],
    scratch_shapes=[pltpu.SMEM((2,), jnp.float32)],
)


def _pass2_body(pf_ref, ge_ref, scores_ref, stats_ref, m_ref, c0_ref,
                wout_ref, bout_ref, out_ref):
    m = stats_ref[0]
    inv_z = 1.0 / stats_ref[1]
    w = jnp.exp(scores_ref[...] - m) * inv_z
    hid = _dot(pf_ref[...], m_ref[...]) + w * ge_ref[...] + c0_ref[...]
    out_ref[...] = _dott(wout_ref[...], hid) + bout_ref[...]


_pass2 = pl.pallas_call(
    _pass2_body,
    grid=(GRID2,),
    in_specs=[
        pl.BlockSpec((B2, D), lambda i: (i, 0)),
        pl.BlockSpec((B2, D), lambda i: (i, 0)),
        pl.BlockSpec((B2, 1), lambda i: (i, 0)),
        pl.BlockSpec(memory_space=pltpu.SMEM),
        pl.BlockSpec((D, D), lambda i: (0, 0)),
        pl.BlockSpec((1, D), lambda i: (0, 0)),
        pl.BlockSpec((K, D), lambda i: (0, 0)),
        pl.BlockSpec((K, 1), lambda i: (0, 0)),
    ],
    out_specs=pl.BlockSpec((K, B2), lambda i: (0, i)),
    out_shape=jax.ShapeDtypeStruct((K, N), jnp.float32),
)


def kernel(product_features, class_tensor, edge_index,
           W_att, b_att, W_pdt, b_pdt, W_comb, b_comb, W_out, b_out):
    e = edge_index[1].astype(jnp.int32)
    e_pad = jnp.pad(e, (0, NPAD - N))
    ct_pad = jnp.pad(class_tensor, ((0, KP - K), (0, 0)))
    s2, g, m, c0 = _prep(ct_pad, W_att, W_pdt, b_pdt.reshape(1, D),
                         W_comb, b_comb.reshape(1, D))
    ge = _make_sc_gather()(e_pad.reshape(NW, NCH, CH), g)
    scores, stats = _pass1(product_features, e.reshape(GRID, 1, B), s2, W_att)
    out_t = _pass2(product_features, ge, scores, stats, m, c0,
                   W_out, b_out.reshape(K, 1))
    return out_t.T


# speedup vs baseline: 1.1384x; 1.1384x over previous
import functools

import jax
import jax.numpy as jnp
from jax import lax
from jax.experimental import pallas as pl
from jax.experimental.pallas import tpu as pltpu
from jax.experimental.pallas import tpu_sc as plsc

N = 50000
D = 128
K = 1000
NW = 32
CH = 112
NCH = 14
PERW = CH * NCH
NPAD = NW * PERW
B = 5000
GRID = N // B
B2 = 1024
GRID2 = 49
NEG = -1e30


def _dot(a, b):
    return lax.dot_general(a, b, (((1,), (0,)), ((), ())),
                           preferred_element_type=jnp.float32)


def _dott(a, b):
    return lax.dot_general(a, b, (((1,), (1,)), ((), ())),
                           preferred_element_type=jnp.float32)


KP = 1024


def _prep_body(ct_ref, watt_ref, wpdt_ref, bpdt_ref, wcomb_ref, bcomb_ref,
               s2_ref, g_ref, m_ref, c0_ref):
    ctp = ct_ref[...]
    ct = ctp[:K, :]
    wcomb = wcomb_ref[...]
    wc_p = wcomb[:, :D]
    wc_c = wcomb[:, D:]
    wa_c = watt_ref[:, D:]
    s_row = _dott(wa_c, ctp)
    for h in range(8):
        s2_ref[h:h + 1, :] = s_row[:, h * 128:(h + 1) * 128]
    g_ref[...] = _dott(ct, wc_c)
    m_ref[...] = lax.dot_general(wpdt_ref[...], wc_p, (((0,), (1,)), ((), ())),
                                 preferred_element_type=jnp.float32)
    c0_ref[...] = _dott(bpdt_ref[...], wc_p) + bcomb_ref[...]


_prep = pl.pallas_call(
    _prep_body,
    in_specs=[
        pl.BlockSpec((KP, D), lambda: (0, 0)),
        pl.BlockSpec((1, 2 * D), lambda: (0, 0)),
        pl.BlockSpec((D, D), lambda: (0, 0)),
        pl.BlockSpec((1, D), lambda: (0, 0)),
        pl.BlockSpec((D, 2 * D), lambda: (0, 0)),
        pl.BlockSpec((1, D), lambda: (0, 0)),
    ],
    out_specs=[
        pl.BlockSpec((8, 128), lambda: (0, 0)),
        pl.BlockSpec((K, D), lambda: (0, 0)),
        pl.BlockSpec((D, D), lambda: (0, 0)),
        pl.BlockSpec((1, D), lambda: (0, 0)),
    ],
    out_shape=[
        jax.ShapeDtypeStruct((8, 128), jnp.float32),
        jax.ShapeDtypeStruct((K, D), jnp.float32),
        jax.ShapeDtypeStruct((D, D), jnp.float32),
        jax.ShapeDtypeStruct((1, D), jnp.float32),
    ],
)


def _sc_gather_body(idx_hbm, tab_hbm, out_hbm, idx_v, rows0, rows1, sem0, sem1):
    wid = lax.axis_index("s") * 2 + lax.axis_index("c")
    base = wid * PERW
    pltpu.sync_copy(idx_hbm.at[wid], idx_v)
    bufs = (rows0, rows1)
    sems = (sem0, sem1)
    copies = [
        pltpu.make_async_copy(tab_hbm.at[idx_v.at[j]], bufs[j % 2], sems[j % 2])
        for j in range(NCH)
    ]
    copies[0].start()
    for j in range(NCH):
        if j + 1 < NCH:
            copies[j + 1].start()
        copies[j].wait()
        pltpu.sync_copy(bufs[j % 2], out_hbm.at[pl.ds(base + j * CH, CH)])


@functools.lru_cache(maxsize=None)
def _make_sc_gather():
    mesh = plsc.VectorSubcoreMesh(core_axis_name="c", subcore_axis_name="s")
    return pl.kernel(
        _sc_gather_body,
        out_type=jax.ShapeDtypeStruct((NPAD, D), jnp.float32),
        mesh=mesh,
        scratch_types=[
            pltpu.VMEM((NCH, CH), jnp.int32),
            pltpu.VMEM((CH, D), jnp.float32),
            pltpu.VMEM((CH, D), jnp.float32),
            pltpu.SemaphoreType.DMA,
            pltpu.SemaphoreType.DMA,
        ],
    )


def _pass1_body(pf_ref, e_ref, s2_ref, watt_ref, scores_ref, stats_ref,
                acc_ref):
    pid = pl.program_id(0)

    @pl.when(pid == 0)
    def _():
        acc_ref[0] = NEG
        acc_ref[1] = 0.0

    pf = pf_ref[...]
    wa_p = watt_ref[:, :D]
    u_col = jnp.sum(pf * wa_p, axis=1, keepdims=True)
    e_row = e_ref[...].reshape(1, B)
    hi = e_row >> 7
    lo = e_row & 127
    oh_hi = (lax.broadcasted_iota(jnp.int32, (8, B), 0) == hi
             ).astype(jnp.float32)
    s_sel = lax.dot_general(s2_ref[...], oh_hi, (((0,), (0,)), ((), ())),
                            preferred_element_type=jnp.float32)
    oh_lo = (lax.broadcasted_iota(jnp.int32, (128, B), 0) == lo
             ).astype(jnp.float32)
    x = oh_lo * s_sel
    ones = jnp.ones((128, 128), jnp.float32)
    s_e = lax.dot_general(x, ones, (((0,), (0,)), ((), ())),
                          preferred_element_type=jnp.float32)
    sc = u_col + s_e[:, :1]
    scores_ref[...] = sc

    m_old = acc_ref[0]
    z_old = acc_ref[1]
    m_new = jnp.maximum(m_old, jnp.max(sc))
    z_new = z_old * jnp.exp(m_old - m_new) + jnp.sum(jnp.exp(sc - m_new))
    acc_ref[0] = m_new
    acc_ref[1] = z_new

    @pl.when(pid == GRID - 1)
    def _():
        stats_ref[0] = m_new
        stats_ref[1] = z_new


_pass1 = pl.pallas_call(
    _pass1_body,
    grid=(GRID,),
    in_specs=[
        pl.BlockSpec((B, D), lambda i: (i, 0)),
        pl.BlockSpec((1, 1, B), lambda i: (i, 0, 0)),
        pl.BlockSpec((8, 128), lambda i: (0, 0)),
        pl.BlockSpec((1, 2 * D), lambda i: (0, 0)),
    ],
    out_specs=[
        pl.BlockSpec((B, 1), lambda i: (i, 0)),
        pl.BlockSpec(memory_space=pltpu.SMEM),
    ],
    out_shape=[
        jax.ShapeDtypeStruct((N, 1), jnp.float32),
        jax.ShapeDtypeStruct((2,), jnp.float32),
    ],
    scratch_shapes=[pltpu.SMEM((2,), jnp.float32)],
)


def _pass2_body(pf_ref, ge_ref, scores_ref, stats_ref, m_ref, c0_ref,
                wout_ref, bout_ref, out_ref):
    m = stats_ref[0]
    inv_z = 1.0 / stats_ref[1]
    w = jnp.exp(scores_ref[...] - m) * inv_z
    hid = _dot(pf_ref[...], m_ref[...]) + w * ge_ref[...] + c0_ref[...]
    out_ref[...] = _dott(wout_ref[...], hid) + bout_ref[...]


_pass2 = pl.pallas_call(
    _pass2_body,
    grid=(GRID2,),
    in_specs=[
        pl.BlockSpec((B2, D), lambda i: (i, 0)),
        pl.BlockSpec((B2, D), lambda i: (i, 0)),
        pl.BlockSpec((B2, 1), lambda i: (i, 0)),
        pl.BlockSpec(memory_space=pltpu.SMEM),
        pl.BlockSpec((D, D), lambda i: (0, 0)),
        pl.BlockSpec((1, D), lambda i: (0, 0)),
        pl.BlockSpec((K, D), lambda i: (0, 0)),
        pl.BlockSpec((K, 1), lambda i: (0, 0)),
    ],
    out_specs=pl.BlockSpec((K, B2), lambda i: (0, i)),
    out_shape=jax.ShapeDtypeStruct((K, N), jnp.float32),
)


def kernel(product_features, class_tensor, edge_index,
           W_att, b_att, W_pdt, b_pdt, W_comb, b_comb, W_out, b_out):
    e = edge_index[1].astype(jnp.int32)
    e_pad = jnp.pad(e, (0, NPAD - N))
    ct_pad = jnp.pad(class_tensor, ((0, KP - K), (0, 0)))
    s2, g, m, c0 = _prep(ct_pad, W_att, W_pdt, b_pdt.reshape(1, D),
                         W_comb, b_comb.reshape(1, D))
    ge = _make_sc_gather()(e_pad.reshape(NW, NCH, CH), g)
    scores, stats = _pass1(product_features, e.reshape(GRID, 1, B), s2, W_att)
    out_t = _pass2(product_features, ge, scores, stats, m, c0,
                   W_out, b_out.reshape(K, 1))
    return out_t.T


# speedup vs baseline: 1.1882x; 1.0437x over previous
import functools

import jax
import jax.numpy as jnp
from jax import lax
from jax.experimental import pallas as pl
from jax.experimental.pallas import tpu as pltpu
from jax.experimental.pallas import tpu_sc as plsc

N = 50000
D = 128
K = 1000
NW = 32
CH = 112
NCH = 14
PERW = CH * NCH
NPAD = NW * PERW
B = 4096
GRID = 13
NP3 = B * GRID
B2 = 1024
GRID2 = 49
NEG = -1e30


def _dot(a, b):
    return lax.dot_general(a, b, (((1,), (0,)), ((), ())),
                           preferred_element_type=jnp.float32)


def _dott(a, b):
    return lax.dot_general(a, b, (((1,), (1,)), ((), ())),
                           preferred_element_type=jnp.float32)


KP = 1024


def _prep_body(ct_ref, watt_ref, wpdt_ref, bpdt_ref, wcomb_ref, bcomb_ref,
               s2_ref, g_ref, m_ref, c0_ref):
    ctp = ct_ref[...]
    ct = ctp[:K, :]
    wcomb = wcomb_ref[...]
    wc_p = wcomb[:, :D]
    wc_c = wcomb[:, D:]
    wa_c = watt_ref[:, D:]
    s_row = _dott(wa_c, ctp)
    for h in range(8):
        s2_ref[h:h + 1, :] = s_row[:, h * 128:(h + 1) * 128]
    g_ref[...] = _dott(ct, wc_c)
    m_ref[...] = lax.dot_general(wpdt_ref[...], wc_p, (((0,), (1,)), ((), ())),
                                 preferred_element_type=jnp.float32)
    c0_ref[...] = _dott(bpdt_ref[...], wc_p) + bcomb_ref[...]


_prep = pl.pallas_call(
    _prep_body,
    in_specs=[
        pl.BlockSpec((KP, D), lambda: (0, 0)),
        pl.BlockSpec((1, 2 * D), lambda: (0, 0)),
        pl.BlockSpec((D, D), lambda: (0, 0)),
        pl.BlockSpec((1, D), lambda: (0, 0)),
        pl.BlockSpec((D, 2 * D), lambda: (0, 0)),
        pl.BlockSpec((1, D), lambda: (0, 0)),
    ],
    out_specs=[
        pl.BlockSpec((8, 128), lambda: (0, 0)),
        pl.BlockSpec((K, D), lambda: (0, 0)),
        pl.BlockSpec((D, D), lambda: (0, 0)),
        pl.BlockSpec((1, D), lambda: (0, 0)),
    ],
    out_shape=[
        jax.ShapeDtypeStruct((8, 128), jnp.float32),
        jax.ShapeDtypeStruct((K, D), jnp.float32),
        jax.ShapeDtypeStruct((D, D), jnp.float32),
        jax.ShapeDtypeStruct((1, D), jnp.float32),
    ],
)


def _sc_gather_body(idx_hbm, tab_hbm, out_hbm, idx_v, rows0, rows1, sem0, sem1):
    wid = lax.axis_index("s") * 2 + lax.axis_index("c")
    base = wid * PERW
    pltpu.sync_copy(idx_hbm.at[wid], idx_v)
    bufs = (rows0, rows1)
    sems = (sem0, sem1)
    copies = [
        pltpu.make_async_copy(tab_hbm.at[idx_v.at[j]], bufs[j % 2], sems[j % 2])
        for j in range(NCH)
    ]
    copies[0].start()
    for j in range(NCH):
        if j + 1 < NCH:
            copies[j + 1].start()
        copies[j].wait()
        pltpu.sync_copy(bufs[j % 2], out_hbm.at[pl.ds(base + j * CH, CH)])


@functools.lru_cache(maxsize=None)
def _make_sc_gather():
    mesh = plsc.VectorSubcoreMesh(core_axis_name="c", subcore_axis_name="s")
    return pl.kernel(
        _sc_gather_body,
        out_type=jax.ShapeDtypeStruct((NPAD, D), jnp.float32),
        mesh=mesh,
        scratch_types=[
            pltpu.VMEM((NCH, CH), jnp.int32),
            pltpu.VMEM((CH, D), jnp.float32),
            pltpu.VMEM((CH, D), jnp.float32),
            pltpu.SemaphoreType.DMA,
            pltpu.SemaphoreType.DMA,
        ],
    )


def _pass1_body(pf_ref, e_ref, s2_ref, watt_ref, scores_ref, stats_ref,
                acc_ref):
    pid = pl.program_id(0)

    @pl.when(pid == 0)
    def _():
        acc_ref[0] = NEG
        acc_ref[1] = 0.0

    pf = pf_ref[...]
    wa_p = watt_ref[:, :D]
    u_t = _dott(wa_p, pf)
    e_row = e_ref[...].reshape(1, B)
    hi = e_row >> 7
    lo = e_row & 127
    oh_hi = (lax.broadcasted_iota(jnp.int32, (8, B), 0) == hi
             ).astype(jnp.float32)
    s_sel = lax.dot_general(s2_ref[...], oh_hi, (((0,), (0,)), ((), ())),
                            preferred_element_type=jnp.float32)
    oh_lo = (lax.broadcasted_iota(jnp.int32, (128, B), 0) == lo
             ).astype(jnp.float32)
    sc = u_t + jnp.sum(oh_lo * s_sel, axis=0, keepdims=True)
    idx = pid * B + lax.broadcasted_iota(jnp.int32, (1, B), 1)
    sc = jnp.where(idx < N, sc, NEG)
    scores_ref[...] = sc

    m_old = acc_ref[0]
    z_old = acc_ref[1]
    m_new = jnp.maximum(m_old, jnp.max(sc))
    z_new = z_old * jnp.exp(m_old - m_new) + jnp.sum(jnp.exp(sc - m_new))
    acc_ref[0] = m_new
    acc_ref[1] = z_new

    @pl.when(pid == GRID - 1)
    def _():
        stats_ref[0] = m_new
        stats_ref[1] = z_new


_pass1 = pl.pallas_call(
    _pass1_body,
    grid=(GRID,),
    in_specs=[
        pl.BlockSpec((B, D), lambda i: (i, 0)),
        pl.BlockSpec((1, 1, B), lambda i: (i, 0, 0)),
        pl.BlockSpec((8, 128), lambda i: (0, 0)),
        pl.BlockSpec((1, 2 * D), lambda i: (0, 0)),
    ],
    out_specs=[
        pl.BlockSpec((1, B), lambda i: (0, i)),
        pl.BlockSpec(memory_space=pltpu.SMEM),
    ],
    out_shape=[
        jax.ShapeDtypeStruct((1, NP3), jnp.float32),
        jax.ShapeDtypeStruct((2,), jnp.float32),
    ],
    scratch_shapes=[pltpu.SMEM((2,), jnp.float32)],
)


def _pass2_body(pf_ref, ge_ref, scores_ref, stats_ref, m_ref, c0_ref,
                wout_ref, bout_ref, out_ref):
    m = stats_ref[0]
    inv_z = 1.0 / stats_ref[1]
    w_row = jnp.exp(scores_ref[...] - m) * inv_z
    hid0 = _dot(pf_ref[...], m_ref[...]) + c0_ref[...]
    out_ref[...] = (_dott(wout_ref[...], hid0)
                    + w_row * _dott(wout_ref[...], ge_ref[...])
                    + bout_ref[...])


_pass2 = pl.pallas_call(
    _pass2_body,
    grid=(GRID2,),
    in_specs=[
        pl.BlockSpec((B2, D), lambda i: (i, 0)),
        pl.BlockSpec((B2, D), lambda i: (i, 0)),
        pl.BlockSpec((1, B2), lambda i: (0, i)),
        pl.BlockSpec(memory_space=pltpu.SMEM),
        pl.BlockSpec((D, D), lambda i: (0, 0)),
        pl.BlockSpec((1, D), lambda i: (0, 0)),
        pl.BlockSpec((K, D), lambda i: (0, 0)),
        pl.BlockSpec((K, 1), lambda i: (0, 0)),
    ],
    out_specs=pl.BlockSpec((K, B2), lambda i: (0, i)),
    out_shape=jax.ShapeDtypeStruct((K, N), jnp.float32),
)


def kernel(product_features, class_tensor, edge_index,
           W_att, b_att, W_pdt, b_pdt, W_comb, b_comb, W_out, b_out):
    e = edge_index[1].astype(jnp.int32)
    e_pad = jnp.pad(e, (0, NPAD - N))
    e_pad3 = jnp.pad(e, (0, NP3 - N)).reshape(GRID, 1, B)
    ct_pad = jnp.pad(class_tensor, ((0, KP - K), (0, 0)))
    s2, g, m, c0 = _prep(ct_pad, W_att, W_pdt, b_pdt.reshape(1, D),
                         W_comb, b_comb.reshape(1, D))
    ge = _make_sc_gather()(e_pad.reshape(NW, NCH, CH), g)
    scores, stats = _pass1(product_features, e_pad3, s2, W_att)
    out_t = _pass2(product_features, ge, scores, stats, m, c0,
                   W_out, b_out.reshape(K, 1))
    return out_t.T


# speedup vs baseline: 1.3114x; 1.1037x over previous
import functools

import jax
import jax.numpy as jnp
from jax import lax
from jax.experimental import pallas as pl
from jax.experimental.pallas import tpu as pltpu
from jax.experimental.pallas import tpu_sc as plsc

N = 50000
D = 128
K = 1000
NW = 32
CH = 112
NCH = 14
PERW = CH * NCH
NPAD = NW * PERW
B = 4096
GRID = 13
NP3 = B * GRID
B2 = 2048
GRID2 = 25
NEG = -1e30


def _dot(a, b):
    return lax.dot_general(a, b, (((1,), (0,)), ((), ())),
                           preferred_element_type=jnp.float32)


def _dott(a, b):
    return lax.dot_general(a, b, (((1,), (1,)), ((), ())),
                           preferred_element_type=jnp.float32)


KP = 1024


def _prep_body(ct_ref, watt_ref, wpdt_ref, bpdt_ref, wcomb_ref, bcomb_ref,
               s2_ref, g_ref, m_ref, c0_ref):
    ctp = ct_ref[...]
    ct = ctp[:K, :]
    wcomb = wcomb_ref[...]
    wc_p = wcomb[:, :D]
    wc_c = wcomb[:, D:]
    wa_c = watt_ref[:, D:]
    s_row = _dott(wa_c, ctp)
    for h in range(8):
        s2_ref[h:h + 1, :] = s_row[:, h * 128:(h + 1) * 128]
    g_ref[...] = _dott(ct, wc_c)
    m_ref[...] = lax.dot_general(wpdt_ref[...], wc_p, (((0,), (1,)), ((), ())),
                                 preferred_element_type=jnp.float32)
    c0_ref[...] = _dott(bpdt_ref[...], wc_p) + bcomb_ref[...]


_prep = pl.pallas_call(
    _prep_body,
    in_specs=[
        pl.BlockSpec((KP, D), lambda: (0, 0)),
        pl.BlockSpec((1, 2 * D), lambda: (0, 0)),
        pl.BlockSpec((D, D), lambda: (0, 0)),
        pl.BlockSpec((1, D), lambda: (0, 0)),
        pl.BlockSpec((D, 2 * D), lambda: (0, 0)),
        pl.BlockSpec((1, D), lambda: (0, 0)),
    ],
    out_specs=[
        pl.BlockSpec((8, 128), lambda: (0, 0)),
        pl.BlockSpec((K, D), lambda: (0, 0)),
        pl.BlockSpec((D, D), lambda: (0, 0)),
        pl.BlockSpec((1, D), lambda: (0, 0)),
    ],
    out_shape=[
        jax.ShapeDtypeStruct((8, 128), jnp.float32),
        jax.ShapeDtypeStruct((K, D), jnp.float32),
        jax.ShapeDtypeStruct((D, D), jnp.float32),
        jax.ShapeDtypeStruct((1, D), jnp.float32),
    ],
)


def _sc_gather_body(idx_hbm, tab_hbm, out_hbm, idx_v, rows0, rows1, sem0, sem1):
    wid = lax.axis_index("s") * 2 + lax.axis_index("c")
    base = wid * PERW
    pltpu.sync_copy(idx_hbm.at[wid], idx_v)
    bufs = (rows0, rows1)
    sems = (sem0, sem1)
    copies = [
        pltpu.make_async_copy(tab_hbm.at[idx_v.at[j]], bufs[j % 2], sems[j % 2])
        for j in range(NCH)
    ]
    copies[0].start()
    for j in range(NCH):
        if j + 1 < NCH:
            copies[j + 1].start()
        copies[j].wait()
        pltpu.sync_copy(bufs[j % 2], out_hbm.at[pl.ds(base + j * CH, CH)])


@functools.lru_cache(maxsize=None)
def _make_sc_gather():
    mesh = plsc.VectorSubcoreMesh(core_axis_name="c", subcore_axis_name="s")
    return pl.kernel(
        _sc_gather_body,
        out_type=jax.ShapeDtypeStruct((NPAD, D), jnp.float32),
        mesh=mesh,
        scratch_types=[
            pltpu.VMEM((NCH, CH), jnp.int32),
            pltpu.VMEM((CH, D), jnp.float32),
            pltpu.VMEM((CH, D), jnp.float32),
            pltpu.SemaphoreType.DMA,
            pltpu.SemaphoreType.DMA,
        ],
    )


def _pass1_body(pf_ref, e_ref, s2_ref, watt_ref, scores_ref, stats_ref,
                acc_ref):
    pid = pl.program_id(0)

    @pl.when(pid == 0)
    def _():
        acc_ref[0] = NEG
        acc_ref[1] = 0.0

    pf = pf_ref[...]
    wa_p = watt_ref[:, :D]
    u_t = _dott(wa_p, pf)
    e_row = e_ref[...].reshape(1, B)
    hi = e_row >> 7
    lo = e_row & 127
    oh_hi = (lax.broadcasted_iota(jnp.int32, (8, B), 0) == hi
             ).astype(jnp.float32)
    s_sel = lax.dot_general(s2_ref[...], oh_hi, (((0,), (0,)), ((), ())),
                            preferred_element_type=jnp.float32)
    oh_lo = (lax.broadcasted_iota(jnp.int32, (128, B), 0) == lo
             ).astype(jnp.float32)
    sc = u_t + jnp.sum(oh_lo * s_sel, axis=0, keepdims=True)
    idx = pid * B + lax.broadcasted_iota(jnp.int32, (1, B), 1)
    sc = jnp.where(idx < N, sc, NEG)
    scores_ref[...] = sc

    m_old = acc_ref[0]
    z_old = acc_ref[1]
    m_new = jnp.maximum(m_old, jnp.max(sc))
    z_new = z_old * jnp.exp(m_old - m_new) + jnp.sum(jnp.exp(sc - m_new))
    acc_ref[0] = m_new
    acc_ref[1] = z_new

    @pl.when(pid == GRID - 1)
    def _():
        stats_ref[0] = m_new
        stats_ref[1] = z_new


_pass1 = pl.pallas_call(
    _pass1_body,
    grid=(GRID,),
    in_specs=[
        pl.BlockSpec((B, D), lambda i: (i, 0)),
        pl.BlockSpec((1, 1, B), lambda i: (i, 0, 0)),
        pl.BlockSpec((8, 128), lambda i: (0, 0)),
        pl.BlockSpec((1, 2 * D), lambda i: (0, 0)),
    ],
    out_specs=[
        pl.BlockSpec((1, B), lambda i: (0, i)),
        pl.BlockSpec(memory_space=pltpu.SMEM),
    ],
    out_shape=[
        jax.ShapeDtypeStruct((1, NP3), jnp.float32),
        jax.ShapeDtypeStruct((2,), jnp.float32),
    ],
    scratch_shapes=[pltpu.SMEM((2,), jnp.float32)],
)


def _pass2_body(pf_ref, ge_ref, scores_ref, stats_ref, m_ref, c0_ref,
                wout_ref, bout_ref, out_ref):
    m = stats_ref[0]
    inv_z = 1.0 / stats_ref[1]
    w_row = jnp.exp(scores_ref[...] - m) * inv_z
    hid0 = _dot(pf_ref[...], m_ref[...]) + c0_ref[...]
    out_ref[...] = (_dott(wout_ref[...], hid0)
                    + w_row * _dott(wout_ref[...], ge_ref[...])
                    + bout_ref[...])


_pass2 = pl.pallas_call(
    _pass2_body,
    grid=(GRID2,),
    in_specs=[
        pl.BlockSpec((B2, D), lambda i: (i, 0)),
        pl.BlockSpec((B2, D), lambda i: (i, 0)),
        pl.BlockSpec((1, B2), lambda i: (0, i)),
        pl.BlockSpec(memory_space=pltpu.SMEM),
        pl.BlockSpec((D, D), lambda i: (0, 0)),
        pl.BlockSpec((1, D), lambda i: (0, 0)),
        pl.BlockSpec((K, D), lambda i: (0, 0)),
        pl.BlockSpec((K, 1), lambda i: (0, 0)),
    ],
    out_specs=pl.BlockSpec((K, B2), lambda i: (0, i)),
    out_shape=jax.ShapeDtypeStruct((K, N), jnp.float32),
)


def kernel(product_features, class_tensor, edge_index,
           W_att, b_att, W_pdt, b_pdt, W_comb, b_comb, W_out, b_out):
    e = edge_index[1].astype(jnp.int32)
    e_pad = jnp.pad(e, (0, NPAD - N))
    e_pad3 = jnp.pad(e, (0, NP3 - N)).reshape(GRID, 1, B)
    ct_pad = jnp.pad(class_tensor, ((0, KP - K), (0, 0)))
    s2, g, m, c0 = _prep(ct_pad, W_att, W_pdt, b_pdt.reshape(1, D),
                         W_comb, b_comb.reshape(1, D))
    ge = _make_sc_gather()(e_pad.reshape(NW, NCH, CH), g)
    scores, stats = _pass1(product_features, e_pad3, s2, W_att)
    out_t = _pass2(product_features, ge, scores, stats, m, c0,
                   W_out, b_out.reshape(K, 1))
    return out_t.T


# speedup vs baseline: 1.3444x; 1.0252x over previous
import functools

import jax
import jax.numpy as jnp
from jax import lax
from jax.experimental import pallas as pl
from jax.experimental.pallas import tpu as pltpu
from jax.experimental.pallas import tpu_sc as plsc

N = 50000
D = 128
K = 1000
NW = 32
CH = 112
NCH = 14
PERW = CH * NCH
NPAD = NW * PERW
B = 4096
GRID = 13
NP3 = B * GRID
B2 = 4096
GRID2 = 13
NEG = -1e30


def _dot(a, b):
    return lax.dot_general(a, b, (((1,), (0,)), ((), ())),
                           preferred_element_type=jnp.float32)


def _dott(a, b):
    return lax.dot_general(a, b, (((1,), (1,)), ((), ())),
                           preferred_element_type=jnp.float32)


KP = 1024


def _prep_body(ct_ref, watt_ref, wpdt_ref, bpdt_ref, wcomb_ref, bcomb_ref,
               s2_ref, g_ref, m_ref, c0_ref):
    ctp = ct_ref[...]
    ct = ctp[:K, :]
    wcomb = wcomb_ref[...]
    wc_p = wcomb[:, :D]
    wc_c = wcomb[:, D:]
    wa_c = watt_ref[:, D:]
    s_row = _dott(wa_c, ctp)
    for h in range(8):
        s2_ref[h:h + 1, :] = s_row[:, h * 128:(h + 1) * 128]
    g_ref[...] = _dott(ct, wc_c)
    m_ref[...] = lax.dot_general(wpdt_ref[...], wc_p, (((0,), (1,)), ((), ())),
                                 preferred_element_type=jnp.float32)
    c0_ref[...] = _dott(bpdt_ref[...], wc_p) + bcomb_ref[...]


_prep = pl.pallas_call(
    _prep_body,
    in_specs=[
        pl.BlockSpec((KP, D), lambda: (0, 0)),
        pl.BlockSpec((1, 2 * D), lambda: (0, 0)),
        pl.BlockSpec((D, D), lambda: (0, 0)),
        pl.BlockSpec((1, D), lambda: (0, 0)),
        pl.BlockSpec((D, 2 * D), lambda: (0, 0)),
        pl.BlockSpec((1, D), lambda: (0, 0)),
    ],
    out_specs=[
        pl.BlockSpec((8, 128), lambda: (0, 0)),
        pl.BlockSpec((K, D), lambda: (0, 0)),
        pl.BlockSpec((D, D), lambda: (0, 0)),
        pl.BlockSpec((1, D), lambda: (0, 0)),
    ],
    out_shape=[
        jax.ShapeDtypeStruct((8, 128), jnp.float32),
        jax.ShapeDtypeStruct((K, D), jnp.float32),
        jax.ShapeDtypeStruct((D, D), jnp.float32),
        jax.ShapeDtypeStruct((1, D), jnp.float32),
    ],
)


def _sc_gather_body(idx_hbm, tab_hbm, out_hbm, idx_v, rows0, rows1, sem0, sem1):
    wid = lax.axis_index("s") * 2 + lax.axis_index("c")
    base = wid * PERW
    pltpu.sync_copy(idx_hbm.at[wid], idx_v)
    bufs = (rows0, rows1)
    sems = (sem0, sem1)
    copies = [
        pltpu.make_async_copy(tab_hbm.at[idx_v.at[j]], bufs[j % 2], sems[j % 2])
        for j in range(NCH)
    ]
    copies[0].start()
    for j in range(NCH):
        if j + 1 < NCH:
            copies[j + 1].start()
        copies[j].wait()
        pltpu.sync_copy(bufs[j % 2], out_hbm.at[pl.ds(base + j * CH, CH)])


@functools.lru_cache(maxsize=None)
def _make_sc_gather():
    mesh = plsc.VectorSubcoreMesh(core_axis_name="c", subcore_axis_name="s")
    return pl.kernel(
        _sc_gather_body,
        out_type=jax.ShapeDtypeStruct((NPAD, D), jnp.float32),
        mesh=mesh,
        scratch_types=[
            pltpu.VMEM((NCH, CH), jnp.int32),
            pltpu.VMEM((CH, D), jnp.float32),
            pltpu.VMEM((CH, D), jnp.float32),
            pltpu.SemaphoreType.DMA,
            pltpu.SemaphoreType.DMA,
        ],
    )


def _pass1_body(pf_ref, e_ref, s2_ref, watt_ref, scores_ref, stats_ref,
                acc_ref):
    pid = pl.program_id(0)

    @pl.when(pid == 0)
    def _():
        acc_ref[0] = NEG
        acc_ref[1] = 0.0

    pf = pf_ref[...]
    wa_p = watt_ref[:, :D]
    u_t = _dott(wa_p, pf)
    e_row = e_ref[...].reshape(1, B)
    hi = e_row >> 7
    lo = e_row & 127
    oh_hi = (lax.broadcasted_iota(jnp.int32, (8, B), 0) == hi
             ).astype(jnp.float32)
    s_sel = lax.dot_general(s2_ref[...], oh_hi, (((0,), (0,)), ((), ())),
                            preferred_element_type=jnp.float32)
    oh_lo = (lax.broadcasted_iota(jnp.int32, (128, B), 0) == lo
             ).astype(jnp.float32)
    sc = u_t + jnp.sum(oh_lo * s_sel, axis=0, keepdims=True)
    idx = pid * B + lax.broadcasted_iota(jnp.int32, (1, B), 1)
    sc = jnp.where(idx < N, sc, NEG)
    scores_ref[...] = sc

    m_old = acc_ref[0]
    z_old = acc_ref[1]
    m_new = jnp.maximum(m_old, jnp.max(sc))
    z_new = z_old * jnp.exp(m_old - m_new) + jnp.sum(jnp.exp(sc - m_new))
    acc_ref[0] = m_new
    acc_ref[1] = z_new

    @pl.when(pid == GRID - 1)
    def _():
        stats_ref[0] = m_new
        stats_ref[1] = z_new


_pass1 = pl.pallas_call(
    _pass1_body,
    grid=(GRID,),
    in_specs=[
        pl.BlockSpec((B, D), lambda i: (i, 0)),
        pl.BlockSpec((1, 1, B), lambda i: (i, 0, 0)),
        pl.BlockSpec((8, 128), lambda i: (0, 0)),
        pl.BlockSpec((1, 2 * D), lambda i: (0, 0)),
    ],
    out_specs=[
        pl.BlockSpec((1, B), lambda i: (0, i)),
        pl.BlockSpec(memory_space=pltpu.SMEM),
    ],
    out_shape=[
        jax.ShapeDtypeStruct((1, NP3), jnp.float32),
        jax.ShapeDtypeStruct((2,), jnp.float32),
    ],
    scratch_shapes=[pltpu.SMEM((2,), jnp.float32)],
)


def _pass2_body(pf_ref, ge_ref, scores_ref, stats_ref, m_ref, c0_ref,
                wout_ref, bout_ref, out_ref):
    m = stats_ref[0]
    inv_z = 1.0 / stats_ref[1]
    w_row = jnp.exp(scores_ref[...] - m) * inv_z
    hid0 = _dot(pf_ref[...], m_ref[...]) + c0_ref[...]
    out_ref[...] = (_dott(wout_ref[...], hid0)
                    + w_row * _dott(wout_ref[...], ge_ref[...])
                    + bout_ref[...])


_pass2 = pl.pallas_call(
    _pass2_body,
    grid=(GRID2,),
    in_specs=[
        pl.BlockSpec((B2, D), lambda i: (i, 0)),
        pl.BlockSpec((B2, D), lambda i: (i, 0)),
        pl.BlockSpec((1, B2), lambda i: (0, i)),
        pl.BlockSpec(memory_space=pltpu.SMEM),
        pl.BlockSpec((D, D), lambda i: (0, 0)),
        pl.BlockSpec((1, D), lambda i: (0, 0)),
        pl.BlockSpec((K, D), lambda i: (0, 0)),
        pl.BlockSpec((K, 1), lambda i: (0, 0)),
    ],
    out_specs=pl.BlockSpec((K, B2), lambda i: (0, i)),
    out_shape=jax.ShapeDtypeStruct((K, N), jnp.float32),
)


def kernel(product_features, class_tensor, edge_index,
           W_att, b_att, W_pdt, b_pdt, W_comb, b_comb, W_out, b_out):
    e = edge_index[1].astype(jnp.int32)
    e_pad = jnp.pad(e, (0, NPAD - N))
    e_pad3 = jnp.pad(e, (0, NP3 - N)).reshape(GRID, 1, B)
    ct_pad = jnp.pad(class_tensor, ((0, KP - K), (0, 0)))
    s2, g, m, c0 = _prep(ct_pad, W_att, W_pdt, b_pdt.reshape(1, D),
                         W_comb, b_comb.reshape(1, D))
    ge = _make_sc_gather()(e_pad.reshape(NW, NCH, CH), g)
    scores, stats = _pass1(product_features, e_pad3, s2, W_att)
    out_t = _pass2(product_features, ge, scores, stats, m, c0,
                   W_out, b_out.reshape(K, 1))
    return out_t.T
